# Initial kernel scaffold; baseline (speedup 1.0000x reference)
import jax
import jax.numpy as jnp
from jax.experimental import pallas as pl


def kernel(x, glove_embd):
    raise NotImplementedError("write your pallas kernel here")


# speedup vs baseline: 2.3928x; 2.3928x over previous
import functools

import jax
import jax.numpy as jnp
from jax import lax
from jax.experimental import pallas as pl
from jax.experimental.pallas import tpu as pltpu
from jax.experimental.pallas import tpu_sc as plsc

VOCAB = 100000
EMBED = 100
EMBED_PAD = 128
BATCH = 4096
HIST = 50

NC = 2
NS = 16
NW = NC * NS

B = BATCH * HIST
B_PER_W = B // NW
CHUNK = 128
NCHUNK = B_PER_W // CHUNK
RING = 5


def _make_gather():
    mesh = plsc.VectorSubcoreMesh(core_axis_name="c", subcore_axis_name="s")

    @functools.partial(
        pl.kernel,
        mesh=mesh,
        out_type=jax.ShapeDtypeStruct((B, EMBED_PAD), jnp.float32),
        scratch_types=[
            pltpu.VMEM((NCHUNK, CHUNK), jnp.int32),
            pltpu.VMEM((RING, CHUNK, EMBED_PAD), jnp.float32),
            pltpu.SemaphoreType.DMA((RING,)),
        ],
        compiler_params=pltpu.CompilerParams(use_tc_tiling_on_sc=True),
    )
    def gather_kernel(idx_hbm, table_hbm, out_hbm, idx_v, rows_v, sems):
        wid = lax.axis_index("s") * NC + lax.axis_index("c")
        base = wid * B_PER_W
        pltpu.sync_copy(idx_hbm.at[wid], idx_v)

        def fire(c, b):
            pltpu.async_copy(table_hbm.at[idx_v.at[c]], rows_v.at[b], sems.at[b])

        def drain(b):
            pltpu.make_async_copy(
                table_hbm.at[idx_v.at[0]], rows_v.at[b], sems.at[b]
            ).wait()

        for b in range(RING):
            fire(b, b)

        @pl.loop(0, NCHUNK // RING)
        def _group(i):
            c0 = i * RING
            for b in range(RING):
                c = c0 + b
                drain(b)
                off = pl.multiple_of(base + c * CHUNK, CHUNK)
                pltpu.sync_copy(rows_v.at[b], out_hbm.at[pl.ds(off, CHUNK)])

                @pl.when(c + RING < NCHUNK)
                def _():
                    fire(c + RING, b)

    return gather_kernel


_gather = _make_gather()


def kernel(x, glove_embd):
    idx = x.reshape(NW, NCHUNK, CHUNK).astype(jnp.int32)
    table = jnp.pad(glove_embd, ((0, 0), (0, EMBED_PAD - EMBED)))
    out = _gather(idx, table)
    return out[:, :EMBED].reshape(BATCH, HIST, EMBED)
